# baseline (speedup 1.0000x reference)
import functools

import jax
import jax.numpy as jnp
from jax import lax
from jax.experimental import pallas as pl
from jax.experimental.pallas import tpu as pltpu
from jax.experimental.pallas import tpu_sc as plsc

B = 1024
IN = 768
H = 512
BLK = 256

_SC_INFO = plsc.get_sparse_core_info()
_NC = _SC_INFO.num_cores
_NS = _SC_INFO.num_subcores
_NW = _NC * _NS
_BPW = B // _NW
_LANE = 16
_NCHUNK = H // _LANE


def _encoder_body(x_ref, w1_ref, b1_ref, w2_ref, b2_ref, cb_ref,
                  ze_ref, sc_ref):
    x = x_ref[...]
    h = jnp.maximum(
        jnp.dot(x, w1_ref[...], preferred_element_type=jnp.float32) + b1_ref[...], 0.0)
    z_e = jnp.maximum(
        jnp.dot(h, w2_ref[...], preferred_element_type=jnp.float32) + b2_ref[...], 0.0)
    cb = cb_ref[...]
    rs = jnp.sum(cb, axis=1)[None, :]
    q = jnp.sum(cb * cb, axis=1)[None, :]
    ze_ref[...] = z_e
    sc_ref[...] = jnp.float32(H) * z_e * z_e - 2.0 * z_e * rs + q


def _decoder_body(zq_ref, w3_ref, b3_ref, w4_ref, b4_ref, xr_ref):
    d = jnp.maximum(
        jnp.dot(zq_ref[...], w3_ref[...], preferred_element_type=jnp.float32)
        + b3_ref[...], 0.0)
    logits = jnp.dot(d, w4_ref[...], preferred_element_type=jnp.float32) + b4_ref[...]
    xr_ref[...] = jax.nn.sigmoid(logits)


def _vreg_take(x, idx):
    dnums = lax.GatherDimensionNumbers(
        offset_dims=(), collapsed_slice_dims=(0,), start_index_map=(0,))
    return lax.gather(x, idx[:, None], dnums, slice_sizes=(1,),
                      mode=lax.GatherScatterMode.PROMISE_IN_BOUNDS)


def _vq_sc_body(sc_hbm, cb_hbm, zq_hbm, s_v, idx_v, rows_v, sem):
    wid = lax.axis_index("s") * _NC + lax.axis_index("c")
    base = wid * _BPW
    pltpu.sync_copy(sc_hbm.at[pl.ds(base, _BPW)], s_v)

    lane = lax.iota(jnp.int32, _LANE)

    def one_row(j, acc):
        r = acc[1]
        m = s_v[r, pl.ds(0, _LANE)]
        a = lane
        for c in range(1, _NCHUNK):
            v = s_v[r, pl.ds(c * _LANE, _LANE)]
            i = lane + c * _LANE
            upd = v < m
            m = jnp.where(upd, v, m)
            a = jnp.where(upd, i, a)
        for s in (8, 4, 2, 1):
            perm = lax.bitwise_xor(lane, s)
            mv = _vreg_take(m, perm)
            av = _vreg_take(a, perm)
            pick = (mv < m) | ((mv == m) & (av < a))
            m = jnp.where(pick, mv, m)
            a = jnp.where(pick, av, a)
        return (jnp.where(lane == j, a, acc[0]), r + 1)

    def one_group(g, _):
        acc, _r = lax.fori_loop(0, _LANE, one_row,
                                (jnp.zeros((_LANE,), jnp.int32), g * _LANE))
        idx_v[pl.ds(g * _LANE, _LANE)] = acc
        return 0

    lax.fori_loop(0, _BPW // _LANE, one_group, 0)

    pltpu.async_copy(cb_hbm.at[idx_v], rows_v, sem).wait()
    pltpu.sync_copy(rows_v, zq_hbm.at[pl.ds(base, _BPW)])


def _tc_encoder(x, W1, b1, W2, b2, codebook):
    grid = (B // BLK,)
    full = lambda shape: pl.BlockSpec(shape, lambda i: (0, 0))
    row_blk = lambda cols: pl.BlockSpec((BLK, cols), lambda i: (i, 0))
    return pl.pallas_call(
        _encoder_body,
        grid=grid,
        in_specs=[row_blk(IN), full((IN, H)), full((1, H)), full((H, H)),
                  full((1, H)), full((H, H))],
        out_specs=[row_blk(H), row_blk(H)],
        out_shape=[
            jax.ShapeDtypeStruct((B, H), jnp.float32),
            jax.ShapeDtypeStruct((B, H), jnp.float32),
        ],
        compiler_params=pltpu.CompilerParams(
            dimension_semantics=("arbitrary",)),
    )(x, W1, b1.reshape(1, H), W2, b2.reshape(1, H), codebook)


def _tc_decoder(z_q, W3, b3, W4, b4):
    grid = (B // BLK,)
    full = lambda shape: pl.BlockSpec(shape, lambda i: (0, 0))
    row_blk = lambda cols: pl.BlockSpec((BLK, cols), lambda i: (i, 0))
    return pl.pallas_call(
        _decoder_body,
        grid=grid,
        in_specs=[row_blk(H), full((H, H)), full((1, H)), full((H, IN)),
                  full((1, IN))],
        out_specs=row_blk(IN),
        out_shape=jax.ShapeDtypeStruct((B, IN), jnp.float32),
        compiler_params=pltpu.CompilerParams(
            dimension_semantics=("arbitrary",)),
    )(z_q, W3, b3.reshape(1, H), W4, b4.reshape(1, IN))


_sc_vq = functools.partial(
    pl.kernel,
    out_type=jax.ShapeDtypeStruct((B, H), jnp.float32),
    mesh=plsc.VectorSubcoreMesh(core_axis_name="c", subcore_axis_name="s"),
    scratch_types=[
        pltpu.VMEM((_BPW, H), jnp.float32),
        pltpu.VMEM((_BPW,), jnp.int32),
        pltpu.VMEM((_BPW, H), jnp.float32),
        pltpu.SemaphoreType.DMA,
    ],
)(_vq_sc_body)


@jax.jit
def kernel(x, W1, b1, W2, b2, codebook, W3, b3, W4, b4):
    z_e, scores = _tc_encoder(x, W1, b1, W2, b2, codebook)
    z_q = _sc_vq(scores, codebook)
    x_recon = _tc_decoder(z_q, W3, b3, W4, b4)
    return (x_recon, z_e, z_q)
